# Initial kernel scaffold; baseline (speedup 1.0000x reference)
import jax
import jax.numpy as jnp
from jax.experimental import pallas as pl


def kernel(x, edge_index, edge_weight, W, b):
    raise NotImplementedError("write your pallas kernel here")


# speedup vs baseline: 3.1090x; 3.1090x over previous
import functools

import jax
import jax.numpy as jnp
from jax import lax
from jax.experimental import pallas as pl
from jax.experimental.pallas import tpu as pltpu
from jax.experimental.pallas import tpu_sc as plsc

N_NODES = 10000
IN_F = 128
OUT_F = 128
N_EDGES = 320000

NC = 2
NS = 16
L = 16
NW = NC * NS
FPT = OUT_F // NW
CHUNK = 3200
NCHUNK = N_EDGES // CHUNK
GROUPS = CHUNK // L
TBL = FPT * N_NODES


def _mm_body(x_ref, w_ref, o_ref):
    o_ref[...] = lax.dot_general(
        w_ref[...],
        x_ref[...],
        dimension_numbers=(((0,), (1,)), ((), ())),
        preferred_element_type=jnp.float32,
        precision=lax.Precision.HIGHEST,
    )


def _support_t(x, W):
    n = x.shape[0]
    return pl.pallas_call(
        _mm_body,
        out_shape=jax.ShapeDtypeStruct((OUT_F, n), jnp.float32),
    )(x, W)


_mesh = plsc.VectorSubcoreMesh(
    core_axis_name="c", subcore_axis_name="s", num_cores=NC, num_subcores=NS
)


@functools.partial(
    pl.kernel,
    out_type=jax.ShapeDtypeStruct((OUT_F * N_NODES,), jnp.float32),
    mesh=_mesh,
    compiler_params=pltpu.CompilerParams(needs_layout_passes=False),
    scratch_types=[
        pltpu.VMEM((TBL,), jnp.float32),
        pltpu.VMEM((TBL,), jnp.float32),
        pltpu.VMEM((FPT * L,), jnp.float32),
        pltpu.VMEM((CHUNK,), jnp.int32),
        pltpu.VMEM((CHUNK,), jnp.int32),
        pltpu.VMEM((CHUNK,), jnp.float32),
        pltpu.VMEM((CHUNK,), jnp.int32),
        pltpu.VMEM((CHUNK,), jnp.int32),
        pltpu.VMEM((CHUNK,), jnp.float32),
        pltpu.SemaphoreType.DMA,
        pltpu.SemaphoreType.DMA,
        pltpu.SemaphoreType.DMA,
        pltpu.SemaphoreType.DMA,
        pltpu.SemaphoreType.DMA,
        pltpu.SemaphoreType.DMA,
    ],
)
def _sc_agg(sup_hbm, src_hbm, dst_hbm, ew_hbm, bexp_hbm, out_hbm,
            table_v, acc_v, b_v,
            src0, dst0, ew0, src1, dst1, ew1,
            sem_s0, sem_d0, sem_w0, sem_s1, sem_d1, sem_w1):
    cid = lax.axis_index("c")
    sid = lax.axis_index("s")
    wid = sid * NC + cid
    base = wid * TBL

    pltpu.sync_copy(sup_hbm.at[pl.ds(base, TBL)], table_v)
    pltpu.sync_copy(bexp_hbm.at[pl.ds(wid * FPT * L, FPT * L)], b_v)

    for f in range(FPT):
        bvec = b_v[pl.ds(f * L, L)]

        @pl.loop(0, N_NODES // L)
        def _init(i, f=f, bvec=bvec):
            acc_v[pl.ds(f * N_NODES + i * L, L)] = bvec

    slots = (
        (src0, dst0, ew0, sem_s0, sem_d0, sem_w0),
        (src1, dst1, ew1, sem_s1, sem_d1, sem_w1),
    )

    def start(c, slot):
        s_b, d_b, w_b, s_s, d_s, w_s = slot
        off = c * CHUNK
        pltpu.make_async_copy(src_hbm.at[pl.ds(off, CHUNK)], s_b, s_s).start()
        pltpu.make_async_copy(dst_hbm.at[pl.ds(off, CHUNK)], d_b, d_s).start()
        pltpu.make_async_copy(ew_hbm.at[pl.ds(off, CHUNK)], w_b, w_s).start()

    def wait(slot):
        s_b, d_b, w_b, s_s, d_s, w_s = slot
        pltpu.make_async_copy(src_hbm.at[pl.ds(0, CHUNK)], s_b, s_s).wait()
        pltpu.make_async_copy(dst_hbm.at[pl.ds(0, CHUNK)], d_b, d_s).wait()
        pltpu.make_async_copy(ew_hbm.at[pl.ds(0, CHUNK)], w_b, w_s).wait()

    def process(slot):
        s_b, d_b, w_b = slot[:3]

        @pl.loop(0, GROUPS, unroll=2)
        def _grp(g):
            o = g * L
            s = s_b[pl.ds(o, L)]
            d = d_b[pl.ds(o, L)]
            w = w_b[pl.ds(o, L)]
            for f in range(FPT):
                v = plsc.load_gather(table_v, [s + f * N_NODES])
                plsc.addupdate_scatter(acc_v, [d + f * N_NODES], v * w)

    start(0, slots[0])
    start(1, slots[1])

    @pl.loop(0, NCHUNK, step=2)
    def _chunk(c):
        wait(slots[0])
        process(slots[0])

        @pl.when(c + 2 < NCHUNK)
        def _():
            start(c + 2, slots[0])

        wait(slots[1])
        process(slots[1])

        @pl.when(c + 3 < NCHUNK)
        def _():
            start(c + 3, slots[1])

    pltpu.sync_copy(acc_v, out_hbm.at[pl.ds(base, TBL)])


def kernel(x, edge_index, edge_weight, W, b):
    src = edge_index[0].astype(jnp.int32)
    dst = edge_index[1].astype(jnp.int32)
    support_t = _support_t(x, W)
    b_exp = jnp.broadcast_to(b[:, None], (OUT_F, L)).reshape(-1)
    out_flat = _sc_agg(
        support_t.reshape(-1), src, dst,
        edge_weight.astype(jnp.float32), b_exp,
    )
    return out_flat.reshape(OUT_F, N_NODES).T


# speedup vs baseline: 7.3530x; 2.3651x over previous
import functools

import jax
import jax.numpy as jnp
from jax import lax
from jax.experimental import pallas as pl
from jax.experimental.pallas import tpu as pltpu
from jax.experimental.pallas import tpu_sc as plsc

N_NODES = 10000
IN_F = 128
OUT_F = 128
N_EDGES = 320000

NC = 2
NS = 16
L = 16
NW = NC * NS
FPT = OUT_F // NW
CHUNK = 3200
NCHUNK = N_EDGES // CHUNK
GROUPS = CHUNK // L
TBL = FPT * N_NODES


def _mm_body(x_ref, w_ref, o_ref):
    o_ref[...] = lax.dot_general(
        w_ref[...],
        x_ref[...],
        dimension_numbers=(((0,), (1,)), ((), ())),
        preferred_element_type=jnp.float32,
        precision=lax.Precision.HIGHEST,
    )


def _support_t(x, W):
    n = x.shape[0]
    return pl.pallas_call(
        _mm_body,
        out_shape=jax.ShapeDtypeStruct((OUT_F, n), jnp.float32),
    )(x, W)


_mesh = plsc.VectorSubcoreMesh(
    core_axis_name="c", subcore_axis_name="s", num_cores=NC, num_subcores=NS
)


@functools.partial(
    pl.kernel,
    out_type=jax.ShapeDtypeStruct((OUT_F * N_NODES,), jnp.float32),
    mesh=_mesh,
    compiler_params=pltpu.CompilerParams(needs_layout_passes=False),
    scratch_types=[
        pltpu.VMEM((TBL,), jnp.float32),
        pltpu.VMEM((TBL,), jnp.float32),
        pltpu.VMEM((FPT * L,), jnp.float32),
        pltpu.VMEM((CHUNK,), jnp.int32),
        pltpu.VMEM((CHUNK,), jnp.int32),
        pltpu.VMEM((CHUNK,), jnp.float32),
        pltpu.VMEM((CHUNK,), jnp.int32),
        pltpu.VMEM((CHUNK,), jnp.int32),
        pltpu.VMEM((CHUNK,), jnp.float32),
        pltpu.SemaphoreType.DMA,
        pltpu.SemaphoreType.DMA,
        pltpu.SemaphoreType.DMA,
        pltpu.SemaphoreType.DMA,
        pltpu.SemaphoreType.DMA,
        pltpu.SemaphoreType.DMA,
    ],
)
def _sc_agg(sup_hbm, src_hbm, dst_hbm, ew_hbm, bexp_hbm, out_hbm,
            table_v, acc_v, b_v,
            src0, dst0, ew0, src1, dst1, ew1,
            sem_s0, sem_d0, sem_w0, sem_s1, sem_d1, sem_w1):
    cid = lax.axis_index("c")
    sid = lax.axis_index("s")
    wid = sid * NC + cid
    base = wid * TBL

    pltpu.sync_copy(sup_hbm.at[pl.ds(base, TBL)], table_v)
    pltpu.sync_copy(bexp_hbm.at[pl.ds(wid * FPT * L, FPT * L)], b_v)

    for f in range(FPT):
        bvec = b_v[pl.ds(f * L, L)]

        @pl.loop(0, N_NODES // L)
        def _init(i, f=f, bvec=bvec):
            acc_v[pl.ds(f * N_NODES + i * L, L)] = bvec

    slots = (
        (src0, dst0, ew0, sem_s0, sem_d0, sem_w0),
        (src1, dst1, ew1, sem_s1, sem_d1, sem_w1),
    )

    def start(c, slot):
        s_b, d_b, w_b, s_s, d_s, w_s = slot
        off = c * CHUNK
        pltpu.make_async_copy(src_hbm.at[pl.ds(off, CHUNK)], s_b, s_s).start()
        pltpu.make_async_copy(dst_hbm.at[pl.ds(off, CHUNK)], d_b, d_s).start()
        pltpu.make_async_copy(ew_hbm.at[pl.ds(off, CHUNK)], w_b, w_s).start()

    def wait(slot):
        s_b, d_b, w_b, s_s, d_s, w_s = slot
        pltpu.make_async_copy(src_hbm.at[pl.ds(0, CHUNK)], s_b, s_s).wait()
        pltpu.make_async_copy(dst_hbm.at[pl.ds(0, CHUNK)], d_b, d_s).wait()
        pltpu.make_async_copy(ew_hbm.at[pl.ds(0, CHUNK)], w_b, w_s).wait()

    def process(slot):
        s_b, d_b, w_b = slot[:3]

        @plsc.parallel_loop(0, GROUPS, unroll=8)
        def _grp(g):
            o = g * L
            s = s_b[pl.ds(o, L)]
            d = d_b[pl.ds(o, L)]
            w = w_b[pl.ds(o, L)]
            for f in range(FPT):
                si = s if f == 0 else s + f * N_NODES
                di = d if f == 0 else d + f * N_NODES
                v = plsc.load_gather(table_v, [si])
                plsc.addupdate_scatter(acc_v, [di], v * w)

    start(0, slots[0])
    start(1, slots[1])

    @pl.loop(0, NCHUNK, step=2)
    def _chunk(c):
        wait(slots[0])
        process(slots[0])

        @pl.when(c + 2 < NCHUNK)
        def _():
            start(c + 2, slots[0])

        wait(slots[1])
        process(slots[1])

        @pl.when(c + 3 < NCHUNK)
        def _():
            start(c + 3, slots[1])

    pltpu.sync_copy(acc_v, out_hbm.at[pl.ds(base, TBL)])


def kernel(x, edge_index, edge_weight, W, b):
    src = edge_index[0].astype(jnp.int32)
    dst = edge_index[1].astype(jnp.int32)
    support_t = _support_t(x, W)
    b_exp = jnp.broadcast_to(b[:, None], (OUT_F, L)).reshape(-1)
    out_flat = _sc_agg(
        support_t.reshape(-1), src, dst,
        edge_weight.astype(jnp.float32), b_exp,
    )
    return out_flat.reshape(OUT_F, N_NODES).T
